# baseline (speedup 1.0000x reference)
import functools

import jax
import jax.numpy as jnp
from jax import lax
from jax.experimental import pallas as pl
from jax.experimental.pallas import tpu as pltpu
from jax.experimental.pallas import tpu_sc as plsc

_SEQ = 8192
_DIM = 1024
_BATCH = 4
_TOTAL_ROWS = _BATCH * _SEQ

_Z = 8192
_X = _TOTAL_ROWS - _Z
_NB_FULL = _X // _SEQ
_REM_ROWS = _X - _NB_FULL * _SEQ

_NW = 32
_Q = _Z // _NW
_CH_ROWS = 8
_CH = _CH_ROWS * _DIM
_NCH = _Q // _CH_ROWS
_NBUF = 4

_SBLK = 512


def _tc_body(x_ref, t_ref, o_ref):
    o_ref[...] = x_ref[...] + t_ref[...][None, :, :]


def _tc_full_batches(inputs, pos_table):
    return pl.pallas_call(
        _tc_body,
        grid=(_SEQ // _SBLK, _NB_FULL),
        in_specs=[
            pl.BlockSpec((1, _SBLK, _DIM), lambda s, b: (b, s, 0)),
            pl.BlockSpec((_SBLK, _DIM), lambda s, b: (s, 0)),
        ],
        out_specs=pl.BlockSpec((1, _SBLK, _DIM), lambda s, b: (b, s, 0)),
        out_shape=jax.ShapeDtypeStruct((_NB_FULL, _SEQ, _DIM), jnp.float32),
    )(inputs, pos_table)


def _tc_partial_batch(inputs, pos_table):
    return pl.pallas_call(
        _tc_body,
        grid=(_REM_ROWS // _SBLK,),
        in_specs=[
            pl.BlockSpec((1, _SBLK, _DIM), lambda s: (_NB_FULL, s, 0)),
            pl.BlockSpec((_SBLK, _DIM), lambda s: (s, 0)),
        ],
        out_specs=pl.BlockSpec((1, _SBLK, _DIM), lambda s: (0, s, 0)),
        out_shape=jax.ShapeDtypeStruct((1, _REM_ROWS, _DIM), jnp.float32),
    )(inputs, pos_table)


def _sc_body(x_hbm, t_hbm, o_hbm, *refs):
    ib = refs[0:_NBUF]
    tb = refs[_NBUF:2 * _NBUF]
    ob = refs[2 * _NBUF:3 * _NBUF]
    si = refs[3 * _NBUF:4 * _NBUF]
    st = refs[4 * _NBUF:5 * _NBUF]
    so = refs[5 * _NBUF:6 * _NBUF]

    wid = lax.axis_index("s") * 2 + lax.axis_index("c")
    r0 = _X + wid * _Q

    def start_in(u, c):
        row = r0 + u * _CH_ROWS
        pltpu.async_copy(x_hbm.at[pl.ds(row * _DIM, _CH)], ib[c], si[c])
        pltpu.async_copy(t_hbm.at[pl.ds((row % _SEQ) * _DIM, _CH)], tb[c], st[c])

    def wait_in(c):
        pltpu.make_async_copy(x_hbm.at[pl.ds(0, _CH)], ib[c], si[c]).wait()
        pltpu.make_async_copy(t_hbm.at[pl.ds(0, _CH)], tb[c], st[c]).wait()

    def compute(c):
        src, tsrc, dst = ib[c], tb[c], ob[c]

        @plsc.parallel_loop(0, _CH, step=16, unroll=8)
        def add_body(j):
            sl = pl.ds(j, 16)
            dst[sl] = src[sl] + tsrc[sl]

    def start_out(u, c):
        off = (wid * _Q + u * _CH_ROWS) * _DIM
        pltpu.async_copy(ob[c], o_hbm.at[pl.ds(off, _CH)], so[c])

    def drain_out(c):
        pltpu.make_async_copy(ob[c], o_hbm.at[pl.ds(0, _CH)], so[c]).wait()

    for c in range(_NBUF):
        start_in(c, c)
    for c in range(_NBUF):
        wait_in(c)
        compute(c)
        start_out(c, c)
        start_in(c + _NBUF, c)

    def steady(i, carry):
        for c in range(_NBUF):
            u = i * _NBUF + c
            drain_out(c)
            wait_in(c)
            compute(c)
            start_out(u, c)
            start_in(u + _NBUF, c)
        return carry

    lax.fori_loop(1, _NCH // _NBUF - 1, steady, 0)

    for c in range(_NBUF):
        u = _NCH - _NBUF + c
        drain_out(c)
        wait_in(c)
        compute(c)
        start_out(u, c)
    for c in range(_NBUF):
        drain_out(c)


@functools.partial(
    pl.kernel,
    out_type=jax.ShapeDtypeStruct((_Z * _DIM,), jnp.float32),
    mesh=plsc.VectorSubcoreMesh(core_axis_name="c", subcore_axis_name="s"),
    scratch_types=(
        [pltpu.VMEM((_CH,), jnp.float32)] * (3 * _NBUF)
        + [pltpu.SemaphoreType.DMA] * (3 * _NBUF)
    ),
)
def _sc_add(x_hbm, t_hbm, o_hbm, *refs):
    _sc_body(x_hbm, t_hbm, o_hbm, *refs)


def kernel(inputs, pos_table):
    batch, seq, dim = inputs.shape
    sc_out = _sc_add(inputs.reshape(-1), pos_table.reshape(-1))
    pieces = [_tc_full_batches(inputs, pos_table).reshape(-1)]
    if _REM_ROWS:
        pieces.append(_tc_partial_batch(inputs, pos_table).reshape(-1))
    pieces.append(sc_out)
    return jnp.concatenate(pieces).reshape(batch, seq, dim)


# speedup vs baseline: 2.5293x; 2.5293x over previous
import functools

import jax
import jax.numpy as jnp
from jax import lax
from jax.experimental import pallas as pl
from jax.experimental.pallas import tpu as pltpu
from jax.experimental.pallas import tpu_sc as plsc

_SEQ = 8192
_DIM = 1024
_BATCH = 4
_TOTAL_ROWS = _BATCH * _SEQ

_Z = 8192
_X = _TOTAL_ROWS - _Z
_NB_FULL = _X // _SEQ
_REM_ROWS = _X - _NB_FULL * _SEQ

_NW = 32
_Q = _Z // _NW
_CH_ROWS = 8
_NCH = _Q // _CH_ROWS
_NBUF = 4

_SBLK = 512


def _tc_body(x_ref, t_ref, o_ref):
    o_ref[...] = x_ref[...] + t_ref[...][None, :, :]


def _tc_full_batches(inputs, pos_table):
    return pl.pallas_call(
        _tc_body,
        grid=(_SEQ // _SBLK, _NB_FULL),
        in_specs=[
            pl.BlockSpec((1, _SBLK, _DIM), lambda s, b: (b, s, 0)),
            pl.BlockSpec((_SBLK, _DIM), lambda s, b: (s, 0)),
        ],
        out_specs=pl.BlockSpec((1, _SBLK, _DIM), lambda s, b: (b, s, 0)),
        out_shape=jax.ShapeDtypeStruct((_NB_FULL, _SEQ, _DIM), jnp.float32),
    )(inputs, pos_table)


def _tc_partial_batch(inputs, pos_table):
    return pl.pallas_call(
        _tc_body,
        grid=(_REM_ROWS // _SBLK,),
        in_specs=[
            pl.BlockSpec((1, _SBLK, _DIM), lambda s: (_NB_FULL, s, 0)),
            pl.BlockSpec((_SBLK, _DIM), lambda s: (s, 0)),
        ],
        out_specs=pl.BlockSpec((1, _SBLK, _DIM), lambda s: (0, s, 0)),
        out_shape=jax.ShapeDtypeStruct((1, _REM_ROWS, _DIM), jnp.float32),
    )(inputs, pos_table)


def _sc_body(x_hbm, t_hbm, o_hbm, *refs):
    ib = refs[0:_NBUF]
    tb = refs[_NBUF:2 * _NBUF]
    ob = refs[2 * _NBUF:3 * _NBUF]
    si = refs[3 * _NBUF:4 * _NBUF]
    st = refs[4 * _NBUF:5 * _NBUF]
    so = refs[5 * _NBUF:6 * _NBUF]

    wid = lax.axis_index("s") * 2 + lax.axis_index("c")
    r0 = _X + wid * _Q

    def start_in(u, c):
        row = r0 + u * _CH_ROWS
        pltpu.async_copy(x_hbm.at[pl.ds(row, _CH_ROWS), :], ib[c], si[c])
        pltpu.async_copy(t_hbm.at[pl.ds(row % _SEQ, _CH_ROWS), :], tb[c], st[c])

    def wait_in(c):
        pltpu.make_async_copy(x_hbm.at[pl.ds(0, _CH_ROWS), :], ib[c], si[c]).wait()
        pltpu.make_async_copy(t_hbm.at[pl.ds(0, _CH_ROWS), :], tb[c], st[c]).wait()

    def compute(c):
        src, tsrc, dst = ib[c], tb[c], ob[c]

        @plsc.parallel_loop(0, _DIM, step=16, unroll=2)
        def add_body(j):
            sl = pl.ds(j, 16)
            for r in range(_CH_ROWS):
                dst[r, sl] = src[r, sl] + tsrc[r, sl]

    def start_out(u, c):
        lrow = wid * _Q + u * _CH_ROWS
        pltpu.async_copy(ob[c], o_hbm.at[pl.ds(lrow, _CH_ROWS), :], so[c])

    def drain_out(c):
        pltpu.make_async_copy(ob[c], o_hbm.at[pl.ds(0, _CH_ROWS), :], so[c]).wait()

    for c in range(_NBUF):
        start_in(c, c)
    for c in range(_NBUF):
        wait_in(c)
        compute(c)
        start_out(c, c)
        start_in(c + _NBUF, c)

    def steady(i, carry):
        for c in range(_NBUF):
            u = i * _NBUF + c
            drain_out(c)
            wait_in(c)
            compute(c)
            start_out(u, c)
            start_in(u + _NBUF, c)
        return carry

    lax.fori_loop(1, _NCH // _NBUF - 1, steady, 0)

    for c in range(_NBUF):
        u = _NCH - _NBUF + c
        drain_out(c)
        wait_in(c)
        compute(c)
        start_out(u, c)
    for c in range(_NBUF):
        drain_out(c)


@functools.partial(
    pl.kernel,
    out_type=jax.ShapeDtypeStruct((_Z, _DIM), jnp.float32),
    mesh=plsc.VectorSubcoreMesh(core_axis_name="c", subcore_axis_name="s"),
    scratch_types=(
        [pltpu.VMEM((_CH_ROWS, _DIM), jnp.float32)] * (3 * _NBUF)
        + [pltpu.SemaphoreType.DMA] * (3 * _NBUF)
    ),
    compiler_params=pltpu.CompilerParams(use_tc_tiling_on_sc=True),
)
def _sc_add(x_hbm, t_hbm, o_hbm, *refs):
    _sc_body(x_hbm, t_hbm, o_hbm, *refs)


def kernel(inputs, pos_table):
    batch, seq, dim = inputs.shape
    sc_out = _sc_add(inputs.reshape(batch * seq, dim), pos_table)
    pieces = [_tc_full_batches(inputs, pos_table)]
    if _REM_ROWS:
        pieces.append(_tc_partial_batch(inputs, pos_table))
    pieces.append(sc_out.reshape(1, _Z, dim))
    return jnp.concatenate(pieces, axis=0).reshape(batch, seq, dim)


# speedup vs baseline: 5.7564x; 2.2759x over previous
import jax
import jax.numpy as jnp
from jax.experimental import pallas as pl

_SBLK = 2048


def _add_body(x_ref, t_ref, o_ref):
    o_ref[...] = x_ref[...] + t_ref[...][None, :, :]


def kernel(inputs, pos_table):
    batch, seq, dim = inputs.shape
    return pl.pallas_call(
        _add_body,
        grid=(seq // _SBLK, batch),
        in_specs=[
            pl.BlockSpec((1, _SBLK, dim), lambda s, b: (b, s, 0)),
            pl.BlockSpec((_SBLK, dim), lambda s, b: (s, 0)),
        ],
        out_specs=pl.BlockSpec((1, _SBLK, dim), lambda s, b: (b, s, 0)),
        out_shape=jax.ShapeDtypeStruct((batch, seq, dim), jnp.float32),
    )(inputs, pos_table)
